# baseline (speedup 1.0000x reference)
import functools

import jax
import jax.numpy as jnp
from jax import lax
from jax.experimental import pallas as pl
from jax.experimental.pallas import tpu as pltpu
from jax.experimental.pallas import tpu_sc as plsc

EPS = 1e-12
G = 8
C = 8
M = 16
NG = 512
HALF = 32
BN = 5000


def _gsum():
    r = lax.broadcasted_iota(jnp.int32, (HALF, 4), 0)
    g = lax.broadcasted_iota(jnp.int32, (HALF, 4), 1)
    return (r // C == g).astype(jnp.float32)


def _gboth():
    r = lax.broadcasted_iota(jnp.int32, (HALF, HALF), 0)
    q = lax.broadcasted_iota(jnp.int32, (HALF, HALF), 1)
    return (r // C == q // C).astype(jnp.float32)


def _onehot(v, width):
    i = lax.broadcasted_iota(jnp.int32, (v.shape[0], width), 1)
    return (i == v[:, None]).astype(jnp.float32)


def _layer0_body(x_ref, em0_ref, post_ref, lik_ref):
    oh = _onehot(x_ref[0, 0, :], M)
    joint = jnp.dot(oh, em0_ref[0], preferred_element_type=jnp.float32,
                  precision=lax.Precision.HIGHEST)
    den = jnp.dot(joint, _gsum(), preferred_element_type=jnp.float32,
                  precision=lax.Precision.HIGHEST)
    lik_ref[...] = jnp.log(den + EPS)
    denb = jnp.dot(joint, _gboth(), preferred_element_type=jnp.float32,
                  precision=lax.Precision.HIGHEST)
    post_ref[...] = joint / (denb + EPS)


def _layer_body(agg_ref, x_ref, em_ref, td_ref, post_ref, lik_ref):
    a = agg_ref[...]
    S = _gsum()
    sb = jnp.dot(a, _gboth(), preferred_element_type=jnp.float32,
                  precision=lax.Precision.HIGHEST)
    nrm = jnp.where(sb > 0, a / (sb + EPS), 1.0 / C)
    trans = jnp.dot(nrm, td_ref[0], preferred_element_type=jnp.float32)
    oh = _onehot(x_ref[0, 0, :], M)
    em = jnp.dot(oh, em_ref[0], preferred_element_type=jnp.float32,
                  precision=lax.Precision.HIGHEST)
    joint = em * trans
    den = jnp.dot(joint, S, preferred_element_type=jnp.float32,
                  precision=lax.Precision.HIGHEST)
    lik_ref[...] = jnp.log(den + EPS)
    denb = jnp.dot(joint, _gboth(), preferred_element_type=jnp.float32,
                  precision=lax.Precision.HIGHEST)
    post_ref[...] = joint / (denb + EPS)


def _make_final(nb, bn, lcols):
    def body(lik_ref, b_ref, ct_ref, wt_ref, out_ref, gl_scr):
        i = pl.program_id(0)
        oh = _onehot(b_ref[0, 0, :], NG)
        contrib = lax.dot_general(
            oh, lik_ref[...], (((0,), (0,)), ((), ())),
            preferred_element_type=jnp.float32,
                  precision=lax.Precision.HIGHEST)

        @pl.when(i == 0)
        def _():
            gl_scr[...] = contrib

        @pl.when(i > 0)
        def _():
            gl_scr[...] += contrib

        @pl.when(i == nb - 1)
        def _():
            act = jnp.tanh(jnp.dot(gl_scr[...], ct_ref[...],
                                   preferred_element_type=jnp.float32))
            out_ref[...] = jnp.dot(act, wt_ref[...],
                                   preferred_element_type=jnp.float32)
    return body


def _make_edge_kernel(N, E):
    SUP = 640
    J = SUP // 128
    NSUP = E // SUP
    NT = 16
    ITERS = (NSUP + NT - 1) // NT
    SLAB = -(-(N // NT) // 8) * 8
    LAST = N - (NT - 1) * SLAB
    NPAD = NT * SLAB
    ZR = 136
    ZK = SLAB // ZR
    assert E % SUP == 0 and SLAB % ZR == 0 and LAST % 8 == 0 and LAST > 0
    assert ZR <= SUP

    mesh = plsc.VectorSubcoreMesh(core_axis_name="c", subcore_axis_name="s")

    @functools.partial(
        pl.kernel, mesh=mesh,
        compiler_params=pltpu.CompilerParams(use_tc_tiling_on_sc=False),
        out_type=jax.ShapeDtypeStruct((2 * N, HALF), jnp.float32),
        scratch_types=[
            pltpu.VMEM((SUP,), jnp.int32),
            pltpu.VMEM((J, 128), jnp.int32),
            pltpu.VMEM((SUP, HALF), jnp.float32),
            pltpu.VMEM_SHARED((NPAD, HALF), jnp.float32),
            pltpu.SemaphoreType.DMA,
            pltpu.SemaphoreType.DMA,
        ],
    )
    def edge_kernel(edges, post, agg_out, sidx, dst2d, rows, agg_sh,
                    sem, sem2):
        c = lax.axis_index("c")
        s = lax.axis_index("s")
        base = c * N

        def zb(r, carry):
            rows[r, pl.ds(0, 16)] = jnp.zeros((16,), jnp.float32)
            rows[r, pl.ds(16, 16)] = jnp.zeros((16,), jnp.float32)
            return carry
        lax.fori_loop(0, ZR, zb, None)
        for k in range(ZK):
            pltpu.sync_copy(rows.at[pl.ds(0, ZR)],
                            agg_sh.at[pl.ds(s * SLAB + k * ZR, ZR)])
        plsc.subcore_barrier()

        def super_body(i, carry):
            sc_id = i * NT + s

            @pl.when(sc_id < NSUP)
            def _():
                off = sc_id * SUP
                cp0 = pltpu.async_copy(edges.at[0, pl.ds(off, SUP)], sidx, sem)
                cps = [pltpu.async_copy(edges.at[1, pl.ds(off + j * 128, 128)],
                                        dst2d.at[j], sem) for j in range(J)]
                cp0.wait()
                for cp in cps:
                    cp.wait()

                def gix(k, carry2):
                    o = pl.multiple_of(k * 16, 16)
                    sidx[pl.ds(o, 16)] = sidx[pl.ds(o, 16)] + base
                    return carry2
                lax.fori_loop(0, SUP // 16, gix, None)

                gcs = [pltpu.async_copy(post.at[sidx.at[pl.ds(j * 128, 128)]],
                                        rows.at[pl.ds(j * 128, 128), :], sem)
                       for j in range(J)]
                for cp in gcs:
                    cp.wait()
                scs = [pltpu.async_copy(rows.at[pl.ds(j * 128, 128), :],
                                        agg_sh.at[dst2d.at[j]], sem2, add=True)
                       for j in range(J)]
                for cp in scs:
                    cp.wait()
            return carry
        lax.fori_loop(0, ITERS, super_body, None)

        plsc.subcore_barrier()
        o = s * SLAB

        @pl.when(s < NT - 1)
        def _():
            pltpu.sync_copy(agg_sh.at[pl.ds(o, SLAB)],
                            agg_out.at[pl.ds(base + o, SLAB)])

        @pl.when(s == NT - 1)
        def _():
            pltpu.sync_copy(agg_sh.at[pl.ds(o, LAST)],
                            agg_out.at[pl.ds(base + o, LAST)])

    return edge_kernel


def kernel(x, edge_index, batch, prior, emission, transition, contrastive,
           W_out):
    N = x.shape[0]
    E = edge_index.shape[1]
    L = emission.shape[0]
    CU = contrastive.shape[1]
    NB = N // BN
    assert N % BN == 0

    x = x.astype(jnp.int32)
    batch = batch.astype(jnp.int32)
    edge_index = edge_index.astype(jnp.int32)

    emt = jnp.transpose(emission, (0, 3, 1, 2)).reshape(L, M, G * C)
    em0p = emt[0] * prior.reshape(1, G * C)
    em_tabs = [jnp.stack([t[:, :HALF], t[:, HALF:]])
               for t in [em0p] + [emt[l] for l in range(1, L)]]
    blocks = jnp.transpose(transition, (0, 1, 3, 2))
    eye8 = jnp.eye(G, dtype=jnp.float32)
    td64 = (eye8[None, :, None, :, None]
            * blocks[:, :, :, None, :]).reshape(L, G * C, G * C)
    td_tabs = [jnp.stack([td64[l, :HALF, :HALF], td64[l, HALF:, HALF:]])
               for l in range(L)]
    ct4 = jnp.kron(jnp.eye(L, dtype=jnp.float32), contrastive)
    wt = W_out.T
    x3 = x.reshape(NB, 1, BN)

    node_grid = (2, NB)
    x_spec = pl.BlockSpec((1, 1, BN), lambda h, i: (i, 0, 0))
    tab_spec = lambda r: pl.BlockSpec((1, r, HALF), lambda h, i: (h, 0, 0))
    row_spec = pl.BlockSpec((BN, HALF), lambda h, i: (h * NB + i, 0))
    lik_spec = pl.BlockSpec((BN, 4), lambda h, i: (h * NB + i, 0))

    post, lik0 = pl.pallas_call(
        _layer0_body,
        grid=node_grid,
        in_specs=[x_spec, tab_spec(M)],
        out_specs=[row_spec, lik_spec],
        out_shape=[jax.ShapeDtypeStruct((2 * N, HALF), jnp.float32),
                   jax.ShapeDtypeStruct((2 * N, 4), jnp.float32)],
    )(x3, em_tabs[0])

    edge_kernel = _make_edge_kernel(N, E)
    liks = [lik0]
    for l in range(1, L):
        agg = edge_kernel(edge_index, post)
        post, likl = pl.pallas_call(
            _layer_body,
            grid=node_grid,
            in_specs=[row_spec, x_spec, tab_spec(M), tab_spec(HALF)],
            out_specs=[row_spec, lik_spec],
            out_shape=[jax.ShapeDtypeStruct((2 * N, HALF), jnp.float32),
                       jax.ShapeDtypeStruct((2 * N, 4), jnp.float32)],
        )(agg, x3, em_tabs[l], td_tabs[l])
        liks.append(likl)

    lik_all = jnp.concatenate(
        [jnp.concatenate([lk[:N], lk[N:]], axis=-1) for lk in liks], axis=-1)

    lcols = L * G
    b3 = batch.reshape(NB, 1, BN)
    out = pl.pallas_call(
        _make_final(NB, BN, lcols),
        grid=(NB,),
        in_specs=[pl.BlockSpec((BN, lcols), lambda i: (i, 0)),
                  pl.BlockSpec((1, 1, BN), lambda i: (i, 0, 0)),
                  pl.BlockSpec((lcols, L * CU), lambda i: (0, 0)),
                  pl.BlockSpec((L * CU, W_out.shape[0]), lambda i: (0, 0))],
        out_specs=pl.BlockSpec((NG, W_out.shape[0]), lambda i: (0, 0)),
        out_shape=jax.ShapeDtypeStruct((NG, W_out.shape[0]), jnp.float32),
        scratch_shapes=[pltpu.VMEM((NG, lcols), jnp.float32)],
    )(lik_all, b3, ct4, wt)
    return out


# speedup vs baseline: 1.6148x; 1.6148x over previous
import functools

import jax
import jax.numpy as jnp
from jax import lax
from jax.experimental import pallas as pl
from jax.experimental.pallas import tpu as pltpu
from jax.experimental.pallas import tpu_sc as plsc

EPS = 1e-12
G = 8
C = 8
M = 16
NG = 512
HALF = 32
BR = 1000
RH = 12500

_HI = dict(preferred_element_type=jnp.float32,
           precision=lax.Precision.HIGHEST)
_LO = dict(preferred_element_type=jnp.float32)


def _b4():
    r = lax.broadcasted_iota(jnp.int32, (128, 128), 0)
    q = lax.broadcasted_iota(jnp.int32, (128, 128), 1)
    return (r // C == q // C).astype(jnp.float32)


def _s4():
    r = lax.broadcasted_iota(jnp.int32, (128, 16), 0)
    t = lax.broadcasted_iota(jnp.int32, (128, 16), 1)
    return (r // C == t).astype(jnp.float32)


def _spread():
    u = lax.broadcasted_iota(jnp.int32, (4, 64), 0)
    j = lax.broadcasted_iota(jnp.int32, (4, 64), 1)
    return (j // M == u).astype(jnp.float32)


def _hmask(i, width):
    rows = i * BR + lax.broadcasted_iota(jnp.int32, (BR, width), 0)
    return (rows >= RH).astype(jnp.float32)


def _onehot(v, width):
    i = lax.broadcasted_iota(jnp.int32, (v.shape[0], width), 1)
    return (i == v[:, None]).astype(jnp.float32)


def _oh2(x_ref, i):
    xb = x_ref[0].astype(jnp.float32)
    xs = jnp.dot(xb, _spread(), **_HI)
    j = lax.broadcasted_iota(jnp.int32, (BR, 64), 1)
    oh = ((j % M).astype(jnp.float32) == xs).astype(jnp.float32)
    hm = _hmask(i, 64)
    return jnp.concatenate([oh * (1.0 - hm), oh * hm], axis=1)


def _layer0_body(x_ref, em_ref, post_ref, lik_ref):
    i = pl.program_id(0)
    oh = _oh2(x_ref, i)
    joint = jnp.dot(oh, em_ref[...], **_HI)
    den = jnp.dot(joint, _s4(), **_HI)
    lik_ref[...] = jnp.log(den + EPS)
    denb = jnp.dot(joint, _b4(), **_HI)
    post_ref[...] = joint / (denb + EPS)


def _layer_body(agg_ref, x_ref, em_ref, td_ref, post_ref, lik_ref):
    i = pl.program_id(0)
    a = agg_ref[...]
    B = _b4()
    sb = jnp.dot(a, B, **_HI)
    nrm = jnp.where(sb > 0, a / (sb + EPS), 1.0 / C)
    hm = _hmask(i, 128)
    nrm2 = jnp.concatenate([nrm * (1.0 - hm), nrm * hm], axis=1)
    trans = jnp.dot(nrm2, td_ref[...], **_LO)
    oh = _oh2(x_ref, i)
    em = jnp.dot(oh, em_ref[...], **_HI)
    joint = em * trans
    den = jnp.dot(joint, _s4(), **_HI)
    lik_ref[...] = jnp.log(den + EPS)
    denb = jnp.dot(joint, B, **_HI)
    post_ref[...] = joint / (denb + EPS)


def _make_final(nbr):
    def body(l0_ref, l1_ref, l2_ref, l3_ref, bq_ref, ct_ref, wt_ref,
             out_ref, gl_scr):
        i = pl.program_id(0)
        bq = bq_ref[0]
        hm = _hmask(i, 4)
        liks = [l0_ref[...], l1_ref[...], l2_ref[...], l3_ref[...]]
        total = jnp.zeros((NG, 32), jnp.float32)
        for u in range(4):
            ohu = _onehot(bq[:, u], NG)
            parts = []
            for lk in liks:
                lu = lk[:, u * 4:(u + 1) * 4]
                parts += [lu * (1.0 - hm), lu * hm]
            l2 = jnp.concatenate(parts, axis=1)
            total = total + lax.dot_general(
                ohu, l2, (((0,), (0,)), ((), ())), **_HI)

        @pl.when(i == 0)
        def _():
            gl_scr[...] = total

        @pl.when(i > 0)
        def _():
            gl_scr[...] += total

        @pl.when(i == nbr - 1)
        def _():
            act = jnp.tanh(jnp.dot(gl_scr[...], ct_ref[...], **_LO))
            out_ref[...] = jnp.dot(act, wt_ref[...], **_LO)
    return body


def _make_edge_kernel(N, E):
    SUP = 640
    J = SUP // 128
    NSUP = E // SUP
    NT = 16
    ITERS = (NSUP + NT - 1) // NT
    SLAB = -(-(N // NT) // 8) * 8
    LAST = N - (NT - 1) * SLAB
    NPAD = NT * SLAB
    ZR = 136
    ZK = SLAB // ZR
    assert E % SUP == 0 and SLAB % ZR == 0 and LAST % 8 == 0 and LAST > 0
    assert ZR <= SUP

    mesh = plsc.VectorSubcoreMesh(core_axis_name="c", subcore_axis_name="s")

    @functools.partial(
        pl.kernel, mesh=mesh,
        compiler_params=pltpu.CompilerParams(use_tc_tiling_on_sc=False),
        out_type=jax.ShapeDtypeStruct((2 * N, HALF), jnp.float32),
        scratch_types=[
            pltpu.VMEM((SUP,), jnp.int32),
            pltpu.VMEM((J, 128), jnp.int32),
            pltpu.VMEM((SUP, HALF), jnp.float32),
            pltpu.VMEM_SHARED((NPAD, HALF), jnp.float32),
            pltpu.SemaphoreType.DMA,
            pltpu.SemaphoreType.DMA,
        ],
    )
    def edge_kernel(edges, post, agg_out, sidx, dst2d, rows, agg_sh,
                    sem, sem2):
        c = lax.axis_index("c")
        s = lax.axis_index("s")
        base = c * N

        def zb(r, carry):
            rows[r, pl.ds(0, 16)] = jnp.zeros((16,), jnp.float32)
            rows[r, pl.ds(16, 16)] = jnp.zeros((16,), jnp.float32)
            return carry
        lax.fori_loop(0, ZR, zb, None)
        for k in range(ZK):
            pltpu.sync_copy(rows.at[pl.ds(0, ZR)],
                            agg_sh.at[pl.ds(s * SLAB + k * ZR, ZR)])
        plsc.subcore_barrier()

        def super_body(i, carry):
            sc_id = i * NT + s

            @pl.when(sc_id < NSUP)
            def _():
                off = sc_id * SUP
                cp0 = pltpu.async_copy(edges.at[0, pl.ds(off, SUP)], sidx, sem)
                cps = [pltpu.async_copy(edges.at[1, pl.ds(off + j * 128, 128)],
                                        dst2d.at[j], sem) for j in range(J)]
                cp0.wait()
                for cp in cps:
                    cp.wait()

                def gix(k, carry2):
                    o = pl.multiple_of(k * 16, 16)
                    sidx[pl.ds(o, 16)] = sidx[pl.ds(o, 16)] + base
                    return carry2
                lax.fori_loop(0, SUP // 16, gix, None)

                gcs = [pltpu.async_copy(post.at[sidx.at[pl.ds(j * 128, 128)]],
                                        rows.at[pl.ds(j * 128, 128), :], sem)
                       for j in range(J)]
                for cp in gcs:
                    cp.wait()
                scs = [pltpu.async_copy(rows.at[pl.ds(j * 128, 128), :],
                                        agg_sh.at[dst2d.at[j]], sem2, add=True)
                       for j in range(J)]
                for cp in scs:
                    cp.wait()
            return carry
        lax.fori_loop(0, ITERS, super_body, None)

        plsc.subcore_barrier()
        o = s * SLAB

        @pl.when(s < NT - 1)
        def _():
            pltpu.sync_copy(agg_sh.at[pl.ds(o, SLAB)],
                            agg_out.at[pl.ds(base + o, SLAB)])

        @pl.when(s == NT - 1)
        def _():
            pltpu.sync_copy(agg_sh.at[pl.ds(o, LAST)],
                            agg_out.at[pl.ds(base + o, LAST)])

    return edge_kernel


def kernel(x, edge_index, batch, prior, emission, transition, contrastive,
           W_out):
    N = x.shape[0]
    E = edge_index.shape[1]
    L = emission.shape[0]
    CU = contrastive.shape[1]
    RTOT = 2 * N // 4
    NBR = RTOT // BR
    assert RTOT % BR == 0 and RTOT // 2 == RH

    x = x.astype(jnp.int32)
    batch = batch.astype(jnp.int32)
    edge_index = edge_index.astype(jnp.int32)

    emt = jnp.transpose(emission, (0, 3, 1, 2)).reshape(L, M, G * C)
    em0p = emt[0] * prior.reshape(1, G * C)
    eye4 = jnp.eye(4, dtype=jnp.float32)
    ems = [em0p] + [emt[l] for l in range(1, L)]

    def big_em(t):
        return jnp.concatenate(
            [jnp.kron(eye4, t[:, :HALF]), jnp.kron(eye4, t[:, HALF:])], axis=0)
    em_tabs = [big_em(t) for t in ems]

    blocks = jnp.transpose(transition, (0, 1, 3, 2))
    eye8 = jnp.eye(G, dtype=jnp.float32)
    td64 = (eye8[None, :, None, :, None]
            * blocks[:, :, :, None, :]).reshape(L, G * C, G * C)
    td_tabs = [jnp.concatenate([jnp.kron(eye4, td64[l, :HALF, :HALF]),
                                jnp.kron(eye4, td64[l, HALF:, HALF:])], axis=0)
               for l in range(L)]
    ct4 = jnp.kron(jnp.eye(L, dtype=jnp.float32), contrastive)
    wt = W_out.T

    xr = x.reshape(RH, 4)
    xq3 = jnp.concatenate([xr, xr], axis=0).reshape(NBR, BR, 4)
    br_ = batch.reshape(RH, 4)
    bq3 = jnp.concatenate([br_, br_], axis=0).reshape(NBR, BR, 4)

    grid = (NBR,)
    q_spec = pl.BlockSpec((1, BR, 4), lambda i: (i, 0, 0))
    row_spec = pl.BlockSpec((BR, 128), lambda i: (i, 0))
    lik_spec = pl.BlockSpec((BR, 16), lambda i: (i, 0))
    tab_spec = lambda r: pl.BlockSpec((r, 128), lambda i: (0, 0))
    out_shapes = [jax.ShapeDtypeStruct((RTOT, 128), jnp.float32),
                  jax.ShapeDtypeStruct((RTOT, 16), jnp.float32)]

    post4, lik0 = pl.pallas_call(
        _layer0_body, grid=grid,
        in_specs=[q_spec, tab_spec(128)],
        out_specs=[row_spec, lik_spec],
        out_shape=out_shapes,
    )(xq3, em_tabs[0])

    edge_kernel = _make_edge_kernel(N, E)
    liks = [lik0]
    for l in range(1, L):
        agg = edge_kernel(edge_index, post4.reshape(2 * N, HALF))
        post4, likl = pl.pallas_call(
            _layer_body, grid=grid,
            in_specs=[row_spec, q_spec, tab_spec(128), tab_spec(256)],
            out_specs=[row_spec, lik_spec],
            out_shape=out_shapes,
        )(agg.reshape(RTOT, 128), xq3, em_tabs[l], td_tabs[l])
        liks.append(likl)

    out = pl.pallas_call(
        _make_final(NBR), grid=grid,
        in_specs=[lik_spec, lik_spec, lik_spec, lik_spec, q_spec,
                  pl.BlockSpec((L * G, L * CU), lambda i: (0, 0)),
                  pl.BlockSpec((L * CU, W_out.shape[0]), lambda i: (0, 0))],
        out_specs=pl.BlockSpec((NG, W_out.shape[0]), lambda i: (0, 0)),
        out_shape=jax.ShapeDtypeStruct((NG, W_out.shape[0]), jnp.float32),
        scratch_shapes=[pltpu.VMEM((NG, L * G), jnp.float32)],
    )(liks[0], liks[1], liks[2], liks[3], bq3, ct4, wt)
    return out


# speedup vs baseline: 1.9009x; 1.1772x over previous
import functools

import jax
import jax.numpy as jnp
from jax import lax
from jax.experimental import pallas as pl
from jax.experimental.pallas import tpu as pltpu
from jax.experimental.pallas import tpu_sc as plsc

EPS = 1e-12
G = 8
C = 8
M = 16
NG = 512
HALF = 32
BR = 1000
RH = 12500

_HI = dict(preferred_element_type=jnp.float32,
           precision=lax.Precision.HIGHEST)
_LO = dict(preferred_element_type=jnp.float32)


def _b4():
    r = lax.broadcasted_iota(jnp.int32, (128, 128), 0)
    q = lax.broadcasted_iota(jnp.int32, (128, 128), 1)
    return (r // C == q // C).astype(jnp.float32)


def _s4():
    r = lax.broadcasted_iota(jnp.int32, (128, 16), 0)
    t = lax.broadcasted_iota(jnp.int32, (128, 16), 1)
    return (r // C == t).astype(jnp.float32)


def _spread():
    u = lax.broadcasted_iota(jnp.int32, (4, 64), 0)
    j = lax.broadcasted_iota(jnp.int32, (4, 64), 1)
    return (j // M == u).astype(jnp.float32)


def _hmask(i, width):
    rows = i * BR + lax.broadcasted_iota(jnp.int32, (BR, width), 0)
    return (rows >= RH).astype(jnp.float32)


def _onehot(v, width):
    i = lax.broadcasted_iota(jnp.int32, (v.shape[0], width), 1)
    return (i == v[:, None]).astype(jnp.float32)


def _oh2(x_ref, i):
    xb = x_ref[0].astype(jnp.float32)
    xs = jnp.dot(xb, _spread(), **_HI)
    j = lax.broadcasted_iota(jnp.int32, (BR, 64), 1)
    oh = ((j % M).astype(jnp.float32) == xs).astype(jnp.float32)
    hm = _hmask(i, 64)
    return jnp.concatenate([oh * (1.0 - hm), oh * hm], axis=1)


def _layer0_body(x_ref, em_ref, post_ref, lik_ref):
    i = pl.program_id(0)
    oh = _oh2(x_ref, i)
    joint = jnp.dot(oh, em_ref[...], **_HI)
    den = jnp.dot(joint, _s4(), **_HI)
    lik_ref[...] = jnp.log(den + EPS)
    denb = jnp.dot(joint, _b4(), **_HI)
    post_ref[...] = joint / (denb + EPS)


def _layer_body(agg_ref, x_ref, em_ref, td_ref, post_ref, lik_ref):
    i = pl.program_id(0)
    a = agg_ref[...]
    B = _b4()
    sb = jnp.dot(a, B, **_HI)
    nrm = jnp.where(sb > 0, a / (sb + EPS), 1.0 / C)
    hm = _hmask(i, 128)
    nrm2 = jnp.concatenate([nrm * (1.0 - hm), nrm * hm], axis=1)
    trans = jnp.dot(nrm2, td_ref[...], **_LO)
    oh = _oh2(x_ref, i)
    em = jnp.dot(oh, em_ref[...], **_HI)
    joint = em * trans
    den = jnp.dot(joint, _s4(), **_HI)
    lik_ref[...] = jnp.log(den + EPS)
    denb = jnp.dot(joint, B, **_HI)
    post_ref[...] = joint / (denb + EPS)


def _make_final(nbr):
    def body(l0_ref, l1_ref, l2_ref, l3_ref, bq_ref, ct_ref, wt_ref,
             out_ref, gl_scr):
        i = pl.program_id(0)
        bq = bq_ref[0]
        hm = _hmask(i, 4)
        liks = [l0_ref[...], l1_ref[...], l2_ref[...], l3_ref[...]]
        total = jnp.zeros((NG, 32), jnp.float32)
        for u in range(4):
            ohu = _onehot(bq[:, u], NG)
            parts = []
            for lk in liks:
                lu = lk[:, u * 4:(u + 1) * 4]
                parts += [lu * (1.0 - hm), lu * hm]
            l2 = jnp.concatenate(parts, axis=1)
            total = total + lax.dot_general(
                ohu, l2, (((0,), (0,)), ((), ())), **_HI)

        @pl.when(i == 0)
        def _():
            gl_scr[...] = total

        @pl.when(i > 0)
        def _():
            gl_scr[...] += total

        @pl.when(i == nbr - 1)
        def _():
            act = jnp.tanh(jnp.dot(gl_scr[...], ct_ref[...], **_LO))
            out_ref[...] = jnp.dot(act, wt_ref[...], **_LO)
    return body


def _make_edge_kernel(N, E):
    NCH = E // 128
    NT = 16
    NBUF = 5
    STEPS = -(-(-(-NCH // NT)) // NBUF) * NBUF
    OUTER = STEPS // NBUF
    SLAB = -(-(N // NT) // 8) * 8
    LAST = N - (NT - 1) * SLAB
    NPAD = NT * SLAB
    ZR = 136
    ZK = SLAB // ZR
    assert E % 128 == 0 and SLAB % ZR == 0 and LAST % 8 == 0 and LAST > 0

    mesh = plsc.VectorSubcoreMesh(core_axis_name="c", subcore_axis_name="s")

    @functools.partial(
        pl.kernel, mesh=mesh,
        compiler_params=pltpu.CompilerParams(use_tc_tiling_on_sc=False),
        out_type=jax.ShapeDtypeStruct((2 * N, HALF), jnp.float32),
        scratch_types=[
            pltpu.VMEM((NBUF, 128), jnp.int32),
            pltpu.VMEM((NBUF, 128), jnp.int32),
            pltpu.VMEM((NBUF * 128, HALF), jnp.float32),
            pltpu.VMEM_SHARED((NPAD, HALF), jnp.float32),
        ] + [pltpu.SemaphoreType.DMA] * (3 * NBUF),
    )
    def edge_kernel(edges, post, agg_out, sidx, dst2d, rows, agg_sh, *sems):
        isem = sems[:NBUF]
        gsem = sems[NBUF:2 * NBUF]
        ssem = sems[2 * NBUF:]
        c = lax.axis_index("c")
        s = lax.axis_index("s")
        base = c * N

        def rslot(b):
            return rows.at[pl.ds(b * 128, 128), :]

        def cid(j):
            return s + NT * j

        def issue_idx(j, b):
            o = cid(j) * 128
            pltpu.async_copy(edges.at[0, pl.ds(o, 128)], sidx.at[b], isem[b])
            pltpu.async_copy(edges.at[1, pl.ds(o, 128)], dst2d.at[b], isem[b])

        def wait_idx(j, b):
            o = cid(j) * 128
            pltpu.make_async_copy(edges.at[0, pl.ds(o, 128)], sidx.at[b],
                                  isem[b]).wait()
            pltpu.make_async_copy(edges.at[1, pl.ds(o, 128)], dst2d.at[b],
                                  isem[b]).wait()

        def issue_gather(b):
            for k in range(8):
                sidx[b, pl.ds(k * 16, 16)] = sidx[b, pl.ds(k * 16, 16)] + base
            pltpu.async_copy(post.at[sidx.at[b]], rslot(b), gsem[b])

        def wait_gather(b):
            pltpu.make_async_copy(post.at[sidx.at[b]], rslot(b),
                                  gsem[b]).wait()

        def issue_scatter(b):
            pltpu.async_copy(rslot(b), agg_sh.at[dst2d.at[b]], ssem[b],
                             add=True)

        def wait_scatter(b):
            pltpu.make_async_copy(rslot(b), agg_sh.at[dst2d.at[b]],
                                  ssem[b]).wait()

        def zb(r, carry):
            rows[r, pl.ds(0, 16)] = jnp.zeros((16,), jnp.float32)
            rows[r, pl.ds(16, 16)] = jnp.zeros((16,), jnp.float32)
            return carry
        lax.fori_loop(0, ZR, zb, None)
        for k in range(ZK):
            pltpu.sync_copy(rows.at[pl.ds(0, ZR)],
                            agg_sh.at[pl.ds(s * SLAB + k * ZR, ZR)])
        plsc.subcore_barrier()

        @pl.when(cid(0) < NCH)
        def _():
            issue_idx(0, 0)

        @pl.when(cid(1) < NCH)
        def _():
            issue_idx(1, 1)

        @pl.when(cid(0) < NCH)
        def _():
            wait_idx(0, 0)
            issue_gather(0)

        def outer_body(g, carry):
            for b in range(NBUF):
                j = g * NBUF + b
                b1, b2 = (b + 1) % NBUF, (b + 2) % NBUF

                @pl.when((j >= 3) & (cid(j - 3) < NCH))
                def _():
                    wait_scatter(b2)

                @pl.when(cid(j + 2) < NCH)
                def _():
                    issue_idx(j + 2, b2)

                @pl.when(cid(j + 1) < NCH)
                def _():
                    wait_idx(j + 1, b1)
                    issue_gather(b1)

                @pl.when(cid(j) < NCH)
                def _():
                    wait_gather(b)
                    issue_scatter(b)
            return carry
        lax.fori_loop(0, OUTER, outer_body, None)

        for d in range(3):
            jl = STEPS - 3 + d

            @pl.when(cid(jl) < NCH)
            def _():
                wait_scatter(jl % NBUF)

        plsc.subcore_barrier()
        o = s * SLAB

        @pl.when(s < NT - 1)
        def _():
            pltpu.sync_copy(agg_sh.at[pl.ds(o, SLAB)],
                            agg_out.at[pl.ds(base + o, SLAB)])

        @pl.when(s == NT - 1)
        def _():
            pltpu.sync_copy(agg_sh.at[pl.ds(o, LAST)],
                            agg_out.at[pl.ds(base + o, LAST)])

    return edge_kernel


def kernel(x, edge_index, batch, prior, emission, transition, contrastive,
           W_out):
    N = x.shape[0]
    E = edge_index.shape[1]
    L = emission.shape[0]
    CU = contrastive.shape[1]
    RTOT = 2 * N // 4
    NBR = RTOT // BR
    assert RTOT % BR == 0 and RTOT // 2 == RH

    x = x.astype(jnp.int32)
    batch = batch.astype(jnp.int32)
    edge_index = edge_index.astype(jnp.int32)

    emt = jnp.transpose(emission, (0, 3, 1, 2)).reshape(L, M, G * C)
    em0p = emt[0] * prior.reshape(1, G * C)
    eye4 = jnp.eye(4, dtype=jnp.float32)
    ems = [em0p] + [emt[l] for l in range(1, L)]

    def big_em(t):
        return jnp.concatenate(
            [jnp.kron(eye4, t[:, :HALF]), jnp.kron(eye4, t[:, HALF:])], axis=0)
    em_tabs = [big_em(t) for t in ems]

    blocks = jnp.transpose(transition, (0, 1, 3, 2))
    eye8 = jnp.eye(G, dtype=jnp.float32)
    td64 = (eye8[None, :, None, :, None]
            * blocks[:, :, :, None, :]).reshape(L, G * C, G * C)
    td_tabs = [jnp.concatenate([jnp.kron(eye4, td64[l, :HALF, :HALF]),
                                jnp.kron(eye4, td64[l, HALF:, HALF:])], axis=0)
               for l in range(L)]
    ct4 = jnp.kron(jnp.eye(L, dtype=jnp.float32), contrastive)
    wt = W_out.T

    xr = x.reshape(RH, 4)
    xq3 = jnp.concatenate([xr, xr], axis=0).reshape(NBR, BR, 4)
    br_ = batch.reshape(RH, 4)
    bq3 = jnp.concatenate([br_, br_], axis=0).reshape(NBR, BR, 4)

    grid = (NBR,)
    q_spec = pl.BlockSpec((1, BR, 4), lambda i: (i, 0, 0))
    row_spec = pl.BlockSpec((BR, 128), lambda i: (i, 0))
    lik_spec = pl.BlockSpec((BR, 16), lambda i: (i, 0))
    tab_spec = lambda r: pl.BlockSpec((r, 128), lambda i: (0, 0))
    out_shapes = [jax.ShapeDtypeStruct((RTOT, 128), jnp.float32),
                  jax.ShapeDtypeStruct((RTOT, 16), jnp.float32)]

    post4, lik0 = pl.pallas_call(
        _layer0_body, grid=grid,
        in_specs=[q_spec, tab_spec(128)],
        out_specs=[row_spec, lik_spec],
        out_shape=out_shapes,
    )(xq3, em_tabs[0])

    edge_kernel = _make_edge_kernel(N, E)
    liks = [lik0]
    for l in range(1, L):
        agg = edge_kernel(edge_index, post4.reshape(2 * N, HALF))
        post4, likl = pl.pallas_call(
            _layer_body, grid=grid,
            in_specs=[row_spec, q_spec, tab_spec(128), tab_spec(256)],
            out_specs=[row_spec, lik_spec],
            out_shape=out_shapes,
        )(agg.reshape(RTOT, 128), xq3, em_tabs[l], td_tabs[l])
        liks.append(likl)

    out = pl.pallas_call(
        _make_final(NBR), grid=grid,
        in_specs=[lik_spec, lik_spec, lik_spec, lik_spec, q_spec,
                  pl.BlockSpec((L * G, L * CU), lambda i: (0, 0)),
                  pl.BlockSpec((L * CU, W_out.shape[0]), lambda i: (0, 0))],
        out_specs=pl.BlockSpec((NG, W_out.shape[0]), lambda i: (0, 0)),
        out_shape=jax.ShapeDtypeStruct((NG, W_out.shape[0]), jnp.float32),
        scratch_shapes=[pltpu.VMEM((NG, L * G), jnp.float32)],
    )(liks[0], liks[1], liks[2], liks[3], bq3, ct4, wt)
    return out
